# baseline (speedup 1.0000x reference)
import functools

import jax
import jax.numpy as jnp
from jax import lax
from jax.experimental import pallas as pl
from jax.experimental.pallas import tpu as pltpu
from jax.experimental.pallas import tpu_sc as plsc

_NUM_CODES = 1024
_CODE_DIM = 64
_B = 32768
_BM = 4096
_CW = 0.25
_NCHUNK = 2
_BC = _B // _NCHUNK
_NBC = _BC // _BM


def _vq_tc_kernel(cbt_ref, zt_ref, zsq_ref, wsq_ref, idx_ref, loss_ref):
    cbt = cbt_ref[...]
    zt = zt_ref[...]
    scores = jax.lax.dot_general(
        cbt, zt, (((0,), (0,)), ((), ())),
        preferred_element_type=jnp.float32)
    e = zsq_ref[...] + wsq_ref[...] - 2.0 * scores
    d = jnp.sqrt(jnp.maximum(e, 0.0))
    min_d = jnp.min(d, axis=0, keepdims=True)
    iota_k = jax.lax.broadcasted_iota(jnp.int32, (_NUM_CODES, _BM), 0)
    idx = jnp.min(jnp.where(d == min_d, iota_k, _NUM_CODES), axis=0)
    idx_ref[...] = idx[None, None, :].astype(jnp.int32)
    part = jnp.sum(min_d * min_d)

    @pl.when(pl.program_id(0) == 0)
    def _init():
        loss_ref[...] = part[None, None]

    @pl.when(pl.program_id(0) != 0)
    def _acc():
        loss_ref[...] += part[None, None]


def _tc_part(cbt, zt, z_sq, w_sq, c):
    return pl.pallas_call(
        _vq_tc_kernel,
        grid=(_NBC,),
        in_specs=[
            pl.BlockSpec((_CODE_DIM, _NUM_CODES), lambda i: (0, 0)),
            pl.BlockSpec((_CODE_DIM, _BM), lambda i: (0, c * _NBC + i)),
            pl.BlockSpec((1, _BM), lambda i: (0, c * _NBC + i)),
            pl.BlockSpec((_NUM_CODES, 1), lambda i: (0, 0)),
        ],
        out_specs=[
            pl.BlockSpec((1, 1, _BM), lambda i: (i, 0, 0)),
            pl.BlockSpec((1, 1), lambda i: (0, 0)),
        ],
        out_shape=[
            jax.ShapeDtypeStruct((_NBC, 1, _BM), jnp.int32),
            jax.ShapeDtypeStruct((1, 1), jnp.float32),
        ],
    )(cbt, zt, z_sq, w_sq)


def _make_sc_gather():
    info = plsc.get_sparse_core_info()
    nw = info.num_cores * info.num_subcores
    b_per_w = _BC // nw
    mesh = plsc.VectorSubcoreMesh(core_axis_name="c", subcore_axis_name="s")

    @functools.partial(
        pl.kernel, mesh=mesh,
        compiler_params=pltpu.CompilerParams(use_tc_tiling_on_sc=False),
        out_type=jax.ShapeDtypeStruct((_BC, _CODE_DIM), jnp.float32),
        scratch_types=[
            pltpu.VMEM((b_per_w,), jnp.int32),
            pltpu.VMEM((b_per_w, _CODE_DIM), jnp.float32),
            pltpu.SemaphoreType.DMA,
        ],
    )
    def _gather(cb_hbm, idx_hbm, out_hbm, idx_v, rows_v, sem):
        wid = lax.axis_index("s") * info.num_cores + lax.axis_index("c")
        base = wid * b_per_w
        pltpu.sync_copy(idx_hbm.at[pl.ds(base, b_per_w)], idx_v)
        pltpu.async_copy(cb_hbm.at[idx_v], rows_v, sem).wait()
        pltpu.sync_copy(rows_v, out_hbm.at[pl.ds(base, b_per_w)])

    return _gather


_sc_gather = _make_sc_gather()


def kernel(z, codebook):
    z_sq = jnp.sum(z * z, axis=1)[None, :]
    w_sq = jnp.sum(codebook * codebook, axis=1)[:, None]
    zt = z.T
    cbt = codebook.T
    idx_chunks = []
    zq_chunks = []
    loss = None
    for c in range(_NCHUNK):
        idx3, loss_c = _tc_part(cbt, zt, z_sq, w_sq, c)
        indices_c = idx3.reshape(_BC)
        zq_chunks.append(_sc_gather(codebook, indices_c))
        idx_chunks.append(indices_c)
        loss = loss_c if loss is None else loss + loss_c
    indices = jnp.concatenate(idx_chunks)
    zq = jnp.concatenate(zq_chunks)
    vq_loss = (_CW / (_B * _CODE_DIM)) * loss[0, 0]
    return (zq, indices, vq_loss)


# speedup vs baseline: 1.0691x; 1.0691x over previous
import functools

import jax
import jax.numpy as jnp
from jax import lax
from jax.experimental import pallas as pl
from jax.experimental.pallas import tpu as pltpu
from jax.experimental.pallas import tpu_sc as plsc

_NUM_CODES = 1024
_CODE_DIM = 64
_B = 32768
_BM = 4096
_CW = 0.25
_NCHUNK = 1
_BC = _B // _NCHUNK
_NBC = _BC // _BM


def _vq_tc_kernel(cbt_ref, zt_ref, zsq_ref, wsq_ref, idx_ref, loss_ref):
    cbt = cbt_ref[...]
    zt = zt_ref[...]
    scores = jax.lax.dot_general(
        cbt, zt, (((0,), (0,)), ((), ())),
        preferred_element_type=jnp.float32)
    e = zsq_ref[...] + wsq_ref[...] - 2.0 * scores
    d = jnp.sqrt(jnp.maximum(e, 0.0))
    min_d = jnp.min(d, axis=0, keepdims=True)
    iota_k = jax.lax.broadcasted_iota(jnp.int32, (_NUM_CODES, _BM), 0)
    idx = jnp.min(jnp.where(d == min_d, iota_k, _NUM_CODES), axis=0)
    idx_ref[...] = idx[None, None, :].astype(jnp.int32)
    part = jnp.sum(min_d * min_d)

    @pl.when(pl.program_id(0) == 0)
    def _init():
        loss_ref[...] = part[None, None]

    @pl.when(pl.program_id(0) != 0)
    def _acc():
        loss_ref[...] += part[None, None]


def _tc_part(cbt, zt, z_sq, w_sq, c):
    return pl.pallas_call(
        _vq_tc_kernel,
        grid=(_NBC,),
        in_specs=[
            pl.BlockSpec((_CODE_DIM, _NUM_CODES), lambda i: (0, 0)),
            pl.BlockSpec((_CODE_DIM, _BM), lambda i: (0, c * _NBC + i)),
            pl.BlockSpec((1, _BM), lambda i: (0, c * _NBC + i)),
            pl.BlockSpec((_NUM_CODES, 1), lambda i: (0, 0)),
        ],
        out_specs=[
            pl.BlockSpec((1, 1, _BM), lambda i: (i, 0, 0)),
            pl.BlockSpec((1, 1), lambda i: (0, 0)),
        ],
        out_shape=[
            jax.ShapeDtypeStruct((_NBC, 1, _BM), jnp.int32),
            jax.ShapeDtypeStruct((1, 1), jnp.float32),
        ],
    )(cbt, zt, z_sq, w_sq)


def _make_sc_gather():
    info = plsc.get_sparse_core_info()
    nw = info.num_cores * info.num_subcores
    b_per_w = _BC // nw
    mesh = plsc.VectorSubcoreMesh(core_axis_name="c", subcore_axis_name="s")

    @functools.partial(
        pl.kernel, mesh=mesh,
        compiler_params=pltpu.CompilerParams(use_tc_tiling_on_sc=False),
        out_type=jax.ShapeDtypeStruct((_BC, _CODE_DIM), jnp.float32),
        scratch_types=[
            pltpu.VMEM((b_per_w,), jnp.int32),
            pltpu.VMEM((b_per_w, _CODE_DIM), jnp.float32),
            pltpu.SemaphoreType.DMA,
        ],
    )
    def _gather(cb_hbm, idx_hbm, out_hbm, idx_v, rows_v, sem):
        wid = lax.axis_index("s") * info.num_cores + lax.axis_index("c")
        base = wid * b_per_w
        pltpu.sync_copy(idx_hbm.at[pl.ds(base, b_per_w)], idx_v)
        pltpu.async_copy(cb_hbm.at[idx_v], rows_v, sem).wait()
        pltpu.sync_copy(rows_v, out_hbm.at[pl.ds(base, b_per_w)])

    return _gather


_sc_gather = _make_sc_gather()


def kernel(z, codebook):
    z_sq = jnp.sum(z * z, axis=1)[None, :]
    w_sq = jnp.sum(codebook * codebook, axis=1)[:, None]
    zt = z.T
    cbt = codebook.T
    idx3, loss = _tc_part(cbt, zt, z_sq, w_sq, 0)
    indices = idx3.reshape(_B)
    zq = _sc_gather(codebook, indices)
    vq_loss = (_CW / (_B * _CODE_DIM)) * loss[0, 0]
    return (zq, indices, vq_loss)
